# Initial kernel scaffold; baseline (speedup 1.0000x reference)
import functools

import jax
import jax.numpy as jnp
from jax import lax
from jax.experimental import pallas as pl
from jax.experimental.pallas import tpu as pltpu
from jax.experimental.pallas import tpu_sc as plsc

VOCAB = 100000
D = 64

NC = 2
NS = 16
NW = NC * NS

C = 128
K = 4
ROWS_G = C * K


def _emb_body(nchunks, ngroups, x_ref, tab_ref, out_ref, idx_v, rows_v, sem_g, sem_o):
    wid = lax.axis_index("s") * NC + lax.axis_index("c")
    npw = nchunks * C
    base = wid * npw

    pltpu.sync_copy(x_ref.at[wid], idx_v)

    def group(g, _):
        for j in range(K):
            pltpu.async_copy(tab_ref.at[idx_v.at[g * K + j]],
                             rows_v.at[pl.ds(j * C, C)], sem_g)
        pltpu.make_async_copy(out_ref.at[pl.ds(base + g * ROWS_G, ROWS_G)],
                              rows_v, sem_g).wait()
        pltpu.async_copy(rows_v, out_ref.at[pl.ds(base + g * ROWS_G, ROWS_G)],
                         sem_o).wait()
        return 0

    lax.fori_loop(0, ngroups, group, 0)


def kernel(x, tok_emb_code):
    orig_shape = x.shape
    n = x.size
    assert n % (NW * ROWS_G) == 0
    npw = n // NW
    nchunks = npw // C
    ngroups = nchunks // K

    xw = x.reshape(NW, nchunks, C).astype(jnp.int32)

    mesh = plsc.VectorSubcoreMesh(core_axis_name="c", subcore_axis_name="s")
    k = pl.kernel(
        functools.partial(_emb_body, nchunks, ngroups),
        out_type=jax.ShapeDtypeStruct((n, D), jnp.float32),
        mesh=mesh,
        scratch_types=[
            pltpu.VMEM((nchunks, C), jnp.int32),
            pltpu.VMEM((ROWS_G, D), jnp.float32),
            pltpu.SemaphoreType.DMA,
            pltpu.SemaphoreType.DMA,
        ],
    )
    out = k(xw, tok_emb_code)
    return out.reshape(*orig_shape, D)


# speedup vs baseline: 3.9041x; 3.9041x over previous
import functools

import jax
import jax.numpy as jnp
from jax import lax
from jax.experimental import pallas as pl
from jax.experimental.pallas import tpu as pltpu
from jax.experimental.pallas import tpu_sc as plsc

VOCAB = 100000
D = 64

NC = 2
NS = 16
NW = NC * NS

C = 128
K = 4
ROWS_G = C * K


def _emb_body(nchunks, ngroups, x_ref, tab_ref, out_ref, idx_v, rows_v, sem_g, sem_o):
    wid = lax.axis_index("s") * NC + lax.axis_index("c")
    npw = nchunks * C
    base = wid * npw

    pltpu.sync_copy(x_ref.at[wid], idx_v)

    def group(g, _):
        for j in range(K):
            pltpu.async_copy(tab_ref.at[idx_v.at[g * K + j]],
                             rows_v.at[pl.ds(j * C, C)], sem_g)
        pltpu.make_async_copy(out_ref.at[pl.ds(base + g * ROWS_G, ROWS_G)],
                              rows_v, sem_g).wait()
        pltpu.async_copy(rows_v, out_ref.at[pl.ds(base + g * ROWS_G, ROWS_G)],
                         sem_o).wait()
        return 0

    lax.fori_loop(0, ngroups, group, 0)


def kernel(x, tok_emb_code):
    orig_shape = x.shape
    n = x.size
    assert n % (NW * ROWS_G) == 0
    npw = n // NW
    nchunks = npw // C
    ngroups = nchunks // K

    xw = x.reshape(NW, nchunks, C).astype(jnp.int32)

    mesh = plsc.VectorSubcoreMesh(core_axis_name="c", subcore_axis_name="s")
    k = pl.kernel(
        functools.partial(_emb_body, nchunks, ngroups),
        out_type=jax.ShapeDtypeStruct((n, D), jnp.float32),
        mesh=mesh,
        compiler_params=pltpu.CompilerParams(use_tc_tiling_on_sc=False),
        scratch_types=[
            pltpu.VMEM((nchunks, C), jnp.int32),
            pltpu.VMEM((ROWS_G, D), jnp.float32),
            pltpu.SemaphoreType.DMA,
            pltpu.SemaphoreType.DMA,
        ],
    )
    out = k(xw, tok_emb_code)
    return out.reshape(*orig_shape, D)


# speedup vs baseline: 4.0642x; 1.0410x over previous
import functools

import jax
import jax.numpy as jnp
from jax import lax
from jax.experimental import pallas as pl
from jax.experimental.pallas import tpu as pltpu
from jax.experimental.pallas import tpu_sc as plsc

VOCAB = 100000
D = 64

NC = 2
NS = 16
NW = NC * NS

C = 128
K = 4
ROWS_G = C * K


NBUF = 2


def _emb_body(nchunks, ngroups, x_ref, tab_ref, out_ref, idx_v, rows_v,
              sem_g0, sem_g1, sem_o0, sem_o1):
    sem_g = (sem_g0, sem_g1)
    sem_o = (sem_o0, sem_o1)
    wid = lax.axis_index("s") * NC + lax.axis_index("c")
    npw = nchunks * C
    base = wid * npw

    pltpu.sync_copy(x_ref.at[wid], idx_v)

    def fire_gathers(g, b):
        for j in range(K):
            pltpu.async_copy(tab_ref.at[idx_v.at[g * K + j]],
                             rows_v.at[b].at[pl.ds(j * C, C)], sem_g[b])

    def drain(buf, sem):
        pltpu.make_async_copy(out_ref.at[pl.ds(base, ROWS_G)], buf, sem).wait()

    for b in range(NBUF):
        fire_gathers(b, b)

    nsteps = ngroups // NBUF

    def step(gp, _):
        for b in range(NBUF):
            g = gp * NBUF + b
            drain(rows_v.at[b], sem_g[b])
            pltpu.async_copy(rows_v.at[b],
                             out_ref.at[pl.ds(base + g * ROWS_G, ROWS_G)],
                             sem_o[b])

            @pl.when(gp < nsteps - 1)
            def _():
                drain(rows_v.at[b], sem_o[b])
                fire_gathers(g + NBUF, b)
        return 0

    lax.fori_loop(0, nsteps, step, 0)
    for b in range(NBUF):
        drain(rows_v.at[b], sem_o[b])


def kernel(x, tok_emb_code):
    orig_shape = x.shape
    n = x.size
    assert n % (NW * ROWS_G) == 0
    npw = n // NW
    nchunks = npw // C
    ngroups = nchunks // K

    xw = x.reshape(NW, nchunks, C).astype(jnp.int32)

    mesh = plsc.VectorSubcoreMesh(core_axis_name="c", subcore_axis_name="s")
    k = pl.kernel(
        functools.partial(_emb_body, nchunks, ngroups),
        out_type=jax.ShapeDtypeStruct((n, D), jnp.float32),
        mesh=mesh,
        compiler_params=pltpu.CompilerParams(use_tc_tiling_on_sc=False),
        scratch_types=[
            pltpu.VMEM((nchunks, C), jnp.int32),
            pltpu.VMEM((NBUF, ROWS_G, D), jnp.float32),
            pltpu.SemaphoreType.DMA,
            pltpu.SemaphoreType.DMA,
            pltpu.SemaphoreType.DMA,
            pltpu.SemaphoreType.DMA,
        ],
    )
    out = k(xw, tok_emb_code)
    return out.reshape(*orig_shape, D)
